# baseline (speedup 1.0000x reference)
import functools

import jax
import jax.numpy as jnp
from jax import lax
from jax.experimental import pallas as pl
from jax.experimental.pallas import tpu as pltpu
from jax.experimental.pallas import tpu_sc as plsc

N_R, D = 100000, 128
NC, NS = 2, 16
NW = NC * NS
CH = 128
NCH = 26
B_PER_W = NCH * CH
N_PAD = NW * B_PER_W


@functools.lru_cache(maxsize=1)
def _make_gather():
    mesh = plsc.VectorSubcoreMesh(
        core_axis_name="c", subcore_axis_name="s", num_cores=NC, num_subcores=NS)

    @functools.partial(
        pl.kernel,
        out_type=(jax.ShapeDtypeStruct((N_PAD, D), jnp.float32),
                  jax.ShapeDtypeStruct((N_PAD, D), jnp.float32)),
        mesh=mesh,
        scratch_types=(
            [pltpu.VMEM((NCH, 8, CH), jnp.int32)] * 2
            + [pltpu.VMEM((CH, D), jnp.float32)] * 4
            + [pltpu.SemaphoreType.DMA] * 8
        ),
    )
    def gather_k(tab_u, tab_i, idx_u, idx_i, out_u, out_i,
                 iv_u, iv_i, bu0, bu1, bi0, bi1,
                 gu0, gu1, gi0, gi1, su0, su1, si0, si1):
        wid = lax.axis_index("c") * NS + lax.axis_index("s")
        base = wid * B_PER_W
        pltpu.sync_copy(idx_u.at[wid], iv_u)
        pltpu.sync_copy(idx_i.at[wid], iv_i)
        streams = ((tab_u, iv_u, out_u, (bu0, bu1), (gu0, gu1), (su0, su1)),
                   (tab_i, iv_i, out_i, (bi0, bi1), (gi0, gi1), (si0, si1)))

        def drain_issue(j, b, last):
            for tab, iv, out, bufs, gs, ss in streams:
                pltpu.make_async_copy(tab.at[pl.ds(0, CH)], bufs[b], gs[b]).wait()
                row0 = pl.multiple_of(base + j * CH, CH)
                pltpu.async_copy(bufs[b], out.at[pl.ds(row0, CH)], ss[b])
            if not last:
                for tab, iv, out, bufs, gs, ss in streams:
                    pltpu.make_async_copy(bufs[b], out.at[pl.ds(0, CH)], ss[b]).wait()
                    pltpu.async_copy(tab.at[iv.at[j + 2, 0]], bufs[b], gs[b])

        for b in range(2):
            for tab, iv, out, bufs, gs, ss in streams:
                pltpu.async_copy(tab.at[iv.at[b, 0]], bufs[b], gs[b])

        def body(t, carry):
            for b in range(2):
                drain_issue(2 * t + b, b, False)
            return carry

        lax.fori_loop(0, (NCH - 2) // 2, body, 0)

        for b in range(2):
            drain_issue(NCH - 2 + b, b, True)
        for tab, iv, out, bufs, gs, ss in streams:
            for b in range(2):
                pltpu.make_async_copy(bufs[b], out.at[pl.ds(0, CH)], ss[b]).wait()

    return gather_k


BR = 1000


def _mm_body(r_ref, u_ref, i_ref, w_ref, o_ref):
    acc = jnp.dot(r_ref[...], w_ref[0:D, :], preferred_element_type=jnp.float32)
    acc += jnp.dot(u_ref[...], w_ref[D:2 * D, :], preferred_element_type=jnp.float32)
    acc += jnp.dot(i_ref[...], w_ref[2 * D:3 * D, :], preferred_element_type=jnp.float32)
    o_ref[...] = jnp.maximum(acc, 0.0)


def _matmul_relu(review, ru, ri, w):
    return pl.pallas_call(
        _mm_body,
        grid=(N_R // BR,),
        in_specs=[
            pl.BlockSpec((BR, D), lambda i: (i, 0)),
            pl.BlockSpec((BR, D), lambda i: (i, 0)),
            pl.BlockSpec((BR, D), lambda i: (i, 0)),
            pl.BlockSpec((3 * D, D), lambda i: (0, 0)),
        ],
        out_specs=pl.BlockSpec((BR, D), lambda i: (i, 0)),
        out_shape=jax.ShapeDtypeStruct((N_R, D), jnp.float32),
    )(review, ru, ri, w)


def kernel(review_vecs, user_vecs, item_vecs, review_item_adj, review_user_adj, con_agg_weights):
    perm_i = jax.random.permutation(jax.random.key(1), D)
    perm_u = jax.random.permutation(jax.random.key(2), D)
    wr = con_agg_weights[:D]
    wu = con_agg_weights[D:2 * D][jnp.argsort(perm_u)]
    wi = con_agg_weights[2 * D:][jnp.argsort(perm_i)]
    w = jnp.concatenate([wr, wu, wi], axis=0)

    def pad_idx(a):
        a = jnp.zeros((N_PAD,), jnp.int32).at[:N_R].set(a).reshape(NW, NCH, 1, CH)
        return jnp.pad(a, ((0, 0), (0, 0), (0, 7), (0, 0)))

    gather = _make_gather()
    ru, ri = gather(user_vecs, item_vecs,
                    pad_idx(review_user_adj), pad_idx(review_item_adj))
    return _matmul_relu(review_vecs, ru, ri, w)


# speedup vs baseline: 1.6217x; 1.6217x over previous
import functools

import jax
import jax.numpy as jnp
from jax import lax
from jax.experimental import pallas as pl
from jax.experimental.pallas import tpu as pltpu
from jax.experimental.pallas import tpu_sc as plsc

N_R, D = 100000, 128
NC, NS = 2, 16
NW = NC * NS
CH = 128
NCH = 25
B_PER_W = NCH * CH
N_PAD = NW * B_PER_W
NB = 7


@functools.lru_cache(maxsize=1)
def _make_gather():
    mesh = plsc.VectorSubcoreMesh(
        core_axis_name="c", subcore_axis_name="s", num_cores=NC, num_subcores=NS)

    @functools.partial(
        pl.kernel,
        out_type=(jax.ShapeDtypeStruct((N_PAD, D), jnp.float32),
                  jax.ShapeDtypeStruct((N_PAD, D), jnp.float32)),
        mesh=mesh,
        scratch_types=(
            [pltpu.VMEM((NCH, CH), jnp.int32)] * 2
            + [pltpu.VMEM((CH, D), jnp.float32)] * NB
            + [pltpu.SemaphoreType.DMA] * (2 * NB)
        ),
    )
    def gather_k(tab_u, tab_i, idx_u, idx_i, out_u, out_i, *rest):
        iv_u, iv_i = rest[0], rest[1]
        bufs = rest[2:2 + NB]
        gs = rest[2 + NB:2 + 2 * NB]
        ss = rest[2 + 2 * NB:2 + 3 * NB]
        wid = lax.axis_index("c") * NS + lax.axis_index("s")
        base = wid * B_PER_W
        pltpu.sync_copy(idx_u.at[wid], iv_u)
        pltpu.sync_copy(idx_i.at[wid], iv_i)
        tabs, ivs, outs = (tab_u, tab_i), (iv_u, iv_i), (out_u, out_i)
        NT = 2 * NCH

        def win(w):
            t, j = w & 1, w >> 1
            return tabs[t], ivs[t], outs[t], base + j * CH

        gcp, scp = {}, {}
        waited = set()
        for w in range(min(NB - 1, NT)):
            tab, iv, out, row0 = win(w)
            gcp[w] = pltpu.async_copy(tab.at[iv.at[w >> 1]], bufs[w % NB], gs[w % NB])
        for w in range(NT):
            b = w % NB
            tab, iv, out, row0 = win(w)
            gcp[w].wait()
            scp[w] = pltpu.async_copy(bufs[b], out.at[pl.ds(row0, CH)], ss[b])
            nw = w + NB - 1
            if nw < NT:
                if w >= 1:
                    scp[w - 1].wait()
                    waited.add(w - 1)
                ntab, niv, nout, _ = win(nw)
                gcp[nw] = pltpu.async_copy(
                    ntab.at[niv.at[nw >> 1]], bufs[nw % NB], gs[nw % NB])
        for w in range(NT):
            if w not in waited:
                scp[w].wait()

    return gather_k


BR = 1000


def _mm_body(r_ref, u_ref, i_ref, w_ref, o_ref):
    acc = jnp.dot(r_ref[...], w_ref[0:D, :], preferred_element_type=jnp.float32)
    acc += jnp.dot(u_ref[...], w_ref[D:2 * D, :], preferred_element_type=jnp.float32)
    acc += jnp.dot(i_ref[...], w_ref[2 * D:3 * D, :], preferred_element_type=jnp.float32)
    o_ref[...] = jnp.maximum(acc, 0.0)


def _matmul_relu(review, ru, ri, w):
    return pl.pallas_call(
        _mm_body,
        grid=(N_R // BR,),
        in_specs=[
            pl.BlockSpec((BR, D), lambda i: (i, 0)),
            pl.BlockSpec((BR, D), lambda i: (i, 0)),
            pl.BlockSpec((BR, D), lambda i: (i, 0)),
            pl.BlockSpec((3 * D, D), lambda i: (0, 0)),
        ],
        out_specs=pl.BlockSpec((BR, D), lambda i: (i, 0)),
        out_shape=jax.ShapeDtypeStruct((N_R, D), jnp.float32),
    )(review, ru, ri, w)


def kernel(review_vecs, user_vecs, item_vecs, review_item_adj, review_user_adj, con_agg_weights):
    perm_i = jax.random.permutation(jax.random.key(1), D)
    perm_u = jax.random.permutation(jax.random.key(2), D)
    wr = con_agg_weights[:D]
    wu = con_agg_weights[D:2 * D][jnp.argsort(perm_u)]
    wi = con_agg_weights[2 * D:][jnp.argsort(perm_i)]
    w = jnp.concatenate([wr, wu, wi], axis=0)

    def pad_idx(a):
        return jnp.zeros((N_PAD,), jnp.int32).at[:N_R].set(a).reshape(NW, NCH, CH)

    gather = _make_gather()
    ru, ri = gather(user_vecs, item_vecs,
                    pad_idx(review_user_adj), pad_idx(review_item_adj))
    return _matmul_relu(review_vecs, ru, ri, w)
